# Initial kernel scaffold; baseline (speedup 1.0000x reference)
import jax
import jax.numpy as jnp
from jax.experimental import pallas as pl


def kernel(input_ids, tensor_of_seq_len, temperature, top_k, top_p, emb, W):
    raise NotImplementedError("write your pallas kernel here")


# speedup vs baseline: 22.4197x; 22.4197x over previous
import jax
import jax.numpy as jnp
from jax import lax
from jax.experimental import pallas as pl
from jax.experimental.pallas import tpu as pltpu

_VOCAB = 100000
_DM = 256
_CW = 2048
_NCH = 49
_VPAD = _NCH * _CW
_NEG = float("-inf")
_KEY_NEG_INF = -2139095040
_KEY_POS_INF = 2139095040


def _sortable(x):
    u = lax.bitcast_convert_type(x, jnp.int32)
    return jnp.where(u < 0, jnp.int32(-2147483648) - u, u)


def _search_max_true(pred, lo0, hi0):
    def body(_, carry):
        lo, hi = carry
        mid = (lo >> 1) + (hi >> 1) + ((lo | hi) & 1)
        ok = pred(mid)
        return jnp.where(ok, mid, lo), jnp.where(ok, hi, mid - 1)

    lo, _ = lax.fori_loop(
        0, 32, body, (jnp.int32(lo0), jnp.int32(hi0)), unroll=True)
    return lo


def _select_token(s, g, k, p, vidx):
    keys = _sortable(s)

    def cnt_pred(mid):
        return jnp.sum((keys >= mid).astype(jnp.int32)) >= k

    thr_k = _search_max_true(cnt_pred, _KEY_NEG_INF, _KEY_POS_INF)
    kill_k = (k > 0) & (keys < thr_k)
    s1 = jnp.where(kill_k, _NEG, s)
    keys1 = jnp.where(kill_k, jnp.int32(_KEY_NEG_INF), keys)

    mx = jnp.max(s1)
    e = jnp.exp(s1 - mx)
    z = jnp.sum(e)
    target = (1.0 - p) * z

    def mass_pred(mid):
        return jnp.sum(jnp.where(keys1 <= mid, e, 0.0)) <= target

    thr_p = _search_max_true(mass_pred, _KEY_NEG_INF - 1, _KEY_POS_INF)
    s2 = jnp.where((p < 1.0) & (keys1 <= thr_p), _NEG, s1)

    cand = s2 + g
    m = jnp.max(cand)
    return jnp.min(jnp.where(cand == m, vidx, jnp.int32(_VPAD)))


def _iota_vidx():
    rowi = lax.broadcasted_iota(jnp.int32, (_NCH, _CW), 0)
    coli = lax.broadcasted_iota(jnp.int32, (_NCH, _CW), 1)
    return rowi * _CW + coli


def _step_kernel(row_ref, w_ref, g_ref, temp_ref, k_ref, p_ref,
                 tok_ref, s_scr):
    j = pl.program_id(0)
    chunk = jnp.dot(row_ref[...], w_ref[...],
                    preferred_element_type=jnp.float32)
    s_scr[pl.ds(j, 1), :] = chunk / temp_ref[0, 0]

    @pl.when(j == _NCH - 1)
    def _():
        vidx = _iota_vidx()
        s = jnp.where(vidx < _VOCAB, s_scr[...], _NEG)
        tok_ref[0, 0] = _select_token(s, g_ref[...], k_ref[0, 0],
                                      p_ref[0, 0], vidx)


def _final_kernel(rows_ref, w_ref, g_ref, temp_ref, k_ref, p_ref,
                  out_ref, tok_ref, s_scr):
    j = pl.program_id(0)
    mat = jnp.dot(rows_ref[...], w_ref[...],
                  preferred_element_type=jnp.float32)
    out_ref[...] = mat[:39, :]
    s_scr[pl.ds(j, 1), :] = mat[38:39, :] / temp_ref[0, 0]

    @pl.when(j == _NCH - 1)
    def _():
        vidx = _iota_vidx()
        s = jnp.where(vidx < _VOCAB, s_scr[...], _NEG)
        tok_ref[0, 0] = _select_token(s, g_ref[...], k_ref[0, 0],
                                      p_ref[0, 0], vidx)


_step_call = pl.pallas_call(
    _step_kernel,
    grid=(_NCH,),
    in_specs=[
        pl.BlockSpec((1, _DM), lambda j: (0, 0)),
        pl.BlockSpec((_DM, _CW), lambda j: (0, j)),
        pl.BlockSpec((_NCH, _CW), lambda j: (0, 0)),
        pl.BlockSpec(memory_space=pltpu.SMEM),
        pl.BlockSpec(memory_space=pltpu.SMEM),
        pl.BlockSpec(memory_space=pltpu.SMEM),
    ],
    out_specs=pl.BlockSpec(memory_space=pltpu.SMEM),
    out_shape=jax.ShapeDtypeStruct((1, 1), jnp.int32),
    scratch_shapes=[pltpu.VMEM((_NCH, _CW), jnp.float32)],
)

_final_call = pl.pallas_call(
    _final_kernel,
    grid=(_NCH,),
    in_specs=[
        pl.BlockSpec((40, _DM), lambda j: (0, 0)),
        pl.BlockSpec((_DM, _CW), lambda j: (0, j)),
        pl.BlockSpec((_NCH, _CW), lambda j: (0, 0)),
        pl.BlockSpec(memory_space=pltpu.SMEM),
        pl.BlockSpec(memory_space=pltpu.SMEM),
        pl.BlockSpec(memory_space=pltpu.SMEM),
    ],
    out_specs=[
        pl.BlockSpec((39, _CW), lambda j: (0, j)),
        pl.BlockSpec(memory_space=pltpu.SMEM),
    ],
    out_shape=[
        jax.ShapeDtypeStruct((39, _VOCAB), jnp.float32),
        jax.ShapeDtypeStruct((1, 1), jnp.int32),
    ],
    scratch_shapes=[pltpu.VMEM((_NCH, _CW), jnp.float32)],
)


def kernel(input_ids, tensor_of_seq_len, temperature, top_k, top_p, emb, W):
    gen_len = tensor_of_seq_len.shape[1]

    skey = jax.random.key(42)
    gs = []
    for _ in range(gen_len):
        skey, sub = jax.random.split(skey)
        gs.append(jax.random.gumbel(sub, (1, _VOCAB), jnp.float32))
    g = jnp.concatenate(gs, axis=0)
    g = jnp.pad(g, ((0, 0), (0, _VPAD - _VOCAB)))
    g = g.reshape(gen_len, _NCH, _CW)

    tk = top_k.astype(jnp.int32)
    tokens = []
    cur = input_ids[0, -1]
    for t in range(gen_len - 1):
        row = emb[cur][None, :]
        tok = _step_call(row, W, g[t], temperature, tk, top_p)[0, 0]
        tokens.append(tok)
        cur = tok

    ids39 = jnp.concatenate([input_ids[0], jnp.stack(tokens)], axis=0)
    rows = jnp.concatenate(
        [emb[ids39], jnp.zeros((1, _DM), jnp.float32)], axis=0)
    logits, tok_last = _final_call(rows, W, g[gen_len - 1], temperature,
                                   tk, top_p)
    ids40 = jnp.concatenate([ids39, tok_last[0]], axis=0)
    return ids40[None, :].astype(jnp.int32), logits[None, :, :]


# speedup vs baseline: 33.5454x; 1.4963x over previous
import functools

import jax
import jax.numpy as jnp
import numpy as np
from jax import lax
from jax.experimental import pallas as pl
from jax.experimental.pallas import tpu as pltpu

_VOCAB = 100000
_DM = 256
_CW = 4096
_NCH = 25
_VPAD = _NCH * _CW
_NEG = float("-inf")
_KEY_NEG_INF = -2139095040
_KEY_POS_INF = 2139095040


def _sortable(x):
    u = lax.bitcast_convert_type(x, jnp.int32)
    return jnp.where(u < 0, jnp.int32(-2147483648) - u, u)


def _search_max_true(pred, lo0, hi0):
    def body(_, carry):
        lo, hi = carry
        mid = (lo >> 1) + (hi >> 1) + ((lo | hi) & 1)
        ok = pred(mid)
        return jnp.where(ok, mid, lo), jnp.where(ok, hi, mid - 1)

    lo, _ = lax.fori_loop(
        0, 32, body, (jnp.int32(lo0), jnp.int32(hi0)), unroll=True)
    return lo


def _select_token(s, g, k, p, vidx):
    keys = _sortable(s)

    def cnt_pred(mid):
        return jnp.sum((keys >= mid).astype(jnp.int32)) >= k

    thr_k = _search_max_true(cnt_pred, _KEY_NEG_INF, _KEY_POS_INF)
    kill_k = (k > 0) & (keys < thr_k)
    s1 = jnp.where(kill_k, _NEG, s)
    keys1 = jnp.where(kill_k, jnp.int32(_KEY_NEG_INF), keys)

    mx = jnp.max(s1)
    e = jnp.exp(s1 - mx)
    z = jnp.sum(e)
    target = (1.0 - p) * z

    def mass_pred(mid):
        return jnp.sum(jnp.where(keys1 <= mid, e, 0.0)) <= target

    thr_p = _search_max_true(mass_pred, _KEY_NEG_INF - 1, _KEY_POS_INF)
    s2 = jnp.where((p < 1.0) & (keys1 <= thr_p), _NEG, s1)

    cand = s2 + g
    m = jnp.max(cand)
    return jnp.min(jnp.where(cand == m, vidx, jnp.int32(_VPAD)))


def _iota_vidx():
    rowi = lax.broadcasted_iota(jnp.int32, (_NCH, _CW), 0)
    coli = lax.broadcasted_iota(jnp.int32, (_NCH, _CW), 1)
    return rowi * _CW + coli


def _step_kernel(row_ref, w_ref, g_ref, temp_ref, k_ref, p_ref,
                 tok_ref, s_scr):
    j = pl.program_id(0)
    chunk = jnp.dot(row_ref[...], w_ref[...],
                    preferred_element_type=jnp.float32)
    s_scr[pl.ds(j, 1), :] = chunk / temp_ref[0, 0]

    @pl.when(j == _NCH - 1)
    def _():
        vidx = _iota_vidx()
        s = jnp.where(vidx < _VOCAB, s_scr[...], _NEG)
        tok_ref[0, 0] = _select_token(s, g_ref[...], k_ref[0, 0],
                                      p_ref[0, 0], vidx)


def _final_kernel(rows_ref, w_ref, g_ref, temp_ref, k_ref, p_ref,
                  out_ref, tok_ref, s_scr):
    j = pl.program_id(0)
    mat = jnp.dot(rows_ref[...], w_ref[...],
                  preferred_element_type=jnp.float32)
    out_ref[...] = mat[:39, :]
    s_scr[pl.ds(j, 1), :] = mat[38:39, :] / temp_ref[0, 0]

    @pl.when(j == _NCH - 1)
    def _():
        vidx = _iota_vidx()
        s = jnp.where(vidx < _VOCAB, s_scr[...], _NEG)
        tok_ref[0, 0] = _select_token(s, g_ref[...], k_ref[0, 0],
                                      p_ref[0, 0], vidx)


_step_call = pl.pallas_call(
    _step_kernel,
    grid=(_NCH,),
    in_specs=[
        pl.BlockSpec((1, _DM), lambda j: (0, 0)),
        pl.BlockSpec((_DM, _CW), lambda j: (0, j)),
        pl.BlockSpec((_NCH, _CW), lambda j: (0, 0)),
        pl.BlockSpec(memory_space=pltpu.SMEM),
        pl.BlockSpec(memory_space=pltpu.SMEM),
        pl.BlockSpec(memory_space=pltpu.SMEM),
    ],
    out_specs=pl.BlockSpec(memory_space=pltpu.SMEM),
    out_shape=jax.ShapeDtypeStruct((1, 1), jnp.int32),
    scratch_shapes=[pltpu.VMEM((_NCH, _CW), jnp.float32)],
)

_final_call = pl.pallas_call(
    _final_kernel,
    grid=(_NCH,),
    in_specs=[
        pl.BlockSpec((40, _DM), lambda j: (0, 0)),
        pl.BlockSpec((_DM, _CW), lambda j: (0, j)),
        pl.BlockSpec((_NCH, _CW), lambda j: (0, 0)),
        pl.BlockSpec(memory_space=pltpu.SMEM),
        pl.BlockSpec(memory_space=pltpu.SMEM),
        pl.BlockSpec(memory_space=pltpu.SMEM),
    ],
    out_specs=[
        pl.BlockSpec((39, _CW), lambda j: (0, j)),
        pl.BlockSpec(memory_space=pltpu.SMEM),
    ],
    out_shape=[
        jax.ShapeDtypeStruct((39, _VOCAB), jnp.float32),
        jax.ShapeDtypeStruct((1, 1), jnp.int32),
    ],
    scratch_shapes=[pltpu.VMEM((_NCH, _CW), jnp.float32)],
)


@functools.cache
def _gumbel_table(gen_len):
    with jax.ensure_compile_time_eval():
        skey = jax.random.key(42)
        gs = []
        for _ in range(gen_len):
            skey, sub = jax.random.split(skey)
            gs.append(jax.random.gumbel(sub, (1, _VOCAB), jnp.float32))
        g = jnp.concatenate(gs, axis=0)
        g = jnp.pad(g, ((0, 0), (0, _VPAD - _VOCAB)))
        return np.asarray(g.reshape(gen_len, _NCH, _CW))


def kernel(input_ids, tensor_of_seq_len, temperature, top_k, top_p, emb, W):
    gen_len = tensor_of_seq_len.shape[1]
    g = jnp.asarray(_gumbel_table(gen_len))

    tk = top_k.astype(jnp.int32)
    tokens = []
    cur = input_ids[0, -1]
    for t in range(gen_len - 1):
        row = emb[cur][None, :]
        tok = _step_call(row, W, g[t], temperature, tk, top_p)[0, 0]
        tokens.append(tok)
        cur = tok

    ids39 = jnp.concatenate([input_ids[0], jnp.stack(tokens)], axis=0)
    rows = jnp.concatenate(
        [emb[ids39], jnp.zeros((1, _DM), jnp.float32)], axis=0)
    logits, tok_last = _final_call(rows, W, g[gen_len - 1], temperature,
                                   tk, top_p)
    ids40 = jnp.concatenate([ids39, tok_last[0]], axis=0)
    return ids40[None, :].astype(jnp.int32), logits[None, :, :]


# speedup vs baseline: 35.1606x; 1.0481x over previous
import functools

import jax
import jax.numpy as jnp
import numpy as np
from jax import lax
from jax.experimental import pallas as pl
from jax.experimental.pallas import tpu as pltpu

_VOCAB = 100000
_DM = 256
_CW = 8192
_NCH = 13
_VPAD = _NCH * _CW
_NEG = float("-inf")
_KEY_NEG_INF = -2139095040
_KEY_POS_INF = 2139095040


def _sortable(x):
    u = lax.bitcast_convert_type(x, jnp.int32)
    return jnp.where(u < 0, jnp.int32(-2147483648) - u, u)


def _search_max_true(pred, lo0, hi0):
    def body(_, carry):
        lo, hi = carry
        mid = (lo >> 1) + (hi >> 1) + ((lo | hi) & 1)
        ok = pred(mid)
        return jnp.where(ok, mid, lo), jnp.where(ok, hi, mid - 1)

    lo, _ = lax.fori_loop(
        0, 32, body, (jnp.int32(lo0), jnp.int32(hi0)), unroll=True)
    return lo


def _select_token(s, g, k, p, vidx):
    keys = _sortable(s)

    def cnt_pred(mid):
        return jnp.sum((keys >= mid).astype(jnp.int32)) >= k

    thr_k = _search_max_true(cnt_pred, _KEY_NEG_INF, _KEY_POS_INF)
    kill_k = (k > 0) & (keys < thr_k)
    s1 = jnp.where(kill_k, _NEG, s)
    keys1 = jnp.where(kill_k, jnp.int32(_KEY_NEG_INF), keys)

    mx = jnp.max(s1)
    e = jnp.exp(s1 - mx)
    z = jnp.sum(e)
    target = (1.0 - p) * z

    def mass_pred(mid):
        return jnp.sum(jnp.where(keys1 <= mid, e, 0.0)) <= target

    thr_p = _search_max_true(mass_pred, _KEY_NEG_INF - 1, _KEY_POS_INF)
    s2 = jnp.where((p < 1.0) & (keys1 <= thr_p), _NEG, s1)

    cand = s2 + g
    m = jnp.max(cand)
    return jnp.min(jnp.where(cand == m, vidx, jnp.int32(_VPAD)))


def _iota_vidx():
    rowi = lax.broadcasted_iota(jnp.int32, (_NCH, _CW), 0)
    coli = lax.broadcasted_iota(jnp.int32, (_NCH, _CW), 1)
    return rowi * _CW + coli


def _step_kernel(row_ref, w_ref, g_ref, temp_ref, k_ref, p_ref,
                 tok_ref, s_scr):
    j = pl.program_id(0)
    chunk = jnp.dot(row_ref[...], w_ref[...],
                    preferred_element_type=jnp.float32)
    s_scr[pl.ds(j, 1), :] = chunk / temp_ref[0, 0]

    @pl.when(j == _NCH - 1)
    def _():
        vidx = _iota_vidx()
        s = jnp.where(vidx < _VOCAB, s_scr[...], _NEG)
        tok_ref[0, 0] = _select_token(s, g_ref[...], k_ref[0, 0],
                                      p_ref[0, 0], vidx)


def _final_kernel(rows_ref, w_ref, g_ref, temp_ref, k_ref, p_ref,
                  out_ref, tok_ref, s_scr):
    j = pl.program_id(0)
    mat = jnp.dot(rows_ref[...], w_ref[...],
                  preferred_element_type=jnp.float32)
    out_ref[...] = mat[:39, :]
    s_scr[pl.ds(j, 1), :] = mat[38:39, :] / temp_ref[0, 0]

    @pl.when(j == _NCH - 1)
    def _():
        vidx = _iota_vidx()
        s = jnp.where(vidx < _VOCAB, s_scr[...], _NEG)
        tok_ref[0, 0] = _select_token(s, g_ref[...], k_ref[0, 0],
                                      p_ref[0, 0], vidx)


_step_call = pl.pallas_call(
    _step_kernel,
    grid=(_NCH,),
    in_specs=[
        pl.BlockSpec((1, _DM), lambda j: (0, 0)),
        pl.BlockSpec((_DM, _CW), lambda j: (0, j)),
        pl.BlockSpec((_NCH, _CW), lambda j: (0, 0)),
        pl.BlockSpec(memory_space=pltpu.SMEM),
        pl.BlockSpec(memory_space=pltpu.SMEM),
        pl.BlockSpec(memory_space=pltpu.SMEM),
    ],
    out_specs=pl.BlockSpec(memory_space=pltpu.SMEM),
    out_shape=jax.ShapeDtypeStruct((1, 1), jnp.int32),
    scratch_shapes=[pltpu.VMEM((_NCH, _CW), jnp.float32)],
)

_final_call = pl.pallas_call(
    _final_kernel,
    grid=(_NCH,),
    in_specs=[
        pl.BlockSpec((40, _DM), lambda j: (0, 0)),
        pl.BlockSpec((_DM, _CW), lambda j: (0, j)),
        pl.BlockSpec((_NCH, _CW), lambda j: (0, 0)),
        pl.BlockSpec(memory_space=pltpu.SMEM),
        pl.BlockSpec(memory_space=pltpu.SMEM),
        pl.BlockSpec(memory_space=pltpu.SMEM),
    ],
    out_specs=[
        pl.BlockSpec((39, _CW), lambda j: (0, j)),
        pl.BlockSpec(memory_space=pltpu.SMEM),
    ],
    out_shape=[
        jax.ShapeDtypeStruct((39, _VOCAB), jnp.float32),
        jax.ShapeDtypeStruct((1, 1), jnp.int32),
    ],
    scratch_shapes=[pltpu.VMEM((_NCH, _CW), jnp.float32)],
)


@functools.cache
def _gumbel_table(gen_len):
    with jax.ensure_compile_time_eval():
        skey = jax.random.key(42)
        gs = []
        for _ in range(gen_len):
            skey, sub = jax.random.split(skey)
            gs.append(jax.random.gumbel(sub, (1, _VOCAB), jnp.float32))
        g = jnp.concatenate(gs, axis=0)
        g = jnp.pad(g, ((0, 0), (0, _VPAD - _VOCAB)))
        return np.asarray(g.reshape(gen_len, _NCH, _CW))


def kernel(input_ids, tensor_of_seq_len, temperature, top_k, top_p, emb, W):
    gen_len = tensor_of_seq_len.shape[1]
    g = jnp.asarray(_gumbel_table(gen_len))

    tk = top_k.astype(jnp.int32)
    tokens = []
    cur = input_ids[0, -1]
    for t in range(gen_len - 1):
        row = emb[cur][None, :]
        tok = _step_call(row, W, g[t], temperature, tk, top_p)[0, 0]
        tokens.append(tok)
        cur = tok

    ids39 = jnp.concatenate([input_ids[0], jnp.stack(tokens)], axis=0)
    rows = jnp.concatenate(
        [emb[ids39], jnp.zeros((1, _DM), jnp.float32)], axis=0)
    logits, tok_last = _final_call(rows, W, g[gen_len - 1], temperature,
                                   tk, top_p)
    ids40 = jnp.concatenate([ids39, tok_last[0]], axis=0)
    return ids40[None, :].astype(jnp.int32), logits[None, :, :]


# speedup vs baseline: 38.6545x; 1.0994x over previous
import functools

import jax
import jax.numpy as jnp
import numpy as np
from jax import lax
from jax.experimental import pallas as pl
from jax.experimental.pallas import tpu as pltpu

_VOCAB = 100000
_DM = 256
_CW = 12800
_NCH = 8
_VPAD = _NCH * _CW
_NEG = float("-inf")
_KEY_NEG_INF = -2139095040
_KEY_POS_INF = 2139095040


def _sortable(x):
    u = lax.bitcast_convert_type(x, jnp.int32)
    return jnp.where(u < 0, jnp.int32(-2147483648) - u, u)


def _ceil_avg(a, b):
    return (a >> 1) + (b >> 1) + ((a | b) & 1)


def _search_max_true(pred, lo0, hi0):
    def body(_, carry):
        lo, hi = carry
        m2 = _ceil_avg(lo, hi)
        m1 = _ceil_avg(lo, m2 - 1)
        m3 = _ceil_avg(m2, hi)
        ok1, ok2, ok3 = pred(m1), pred(m2), pred(m3)
        lo2 = jnp.where(ok3, m3, jnp.where(ok2, m2, jnp.where(ok1, m1, lo)))
        hi2 = jnp.where(ok3, hi, jnp.where(ok2, m3 - 1,
                                           jnp.where(ok1, m2 - 1, m1 - 1)))
        return lo2, hi2

    lo, _ = lax.fori_loop(
        0, 16, body, (jnp.int32(lo0), jnp.int32(hi0)), unroll=True)
    return lo


def _select_token(s, g, k, p, vidx):
    keys = _sortable(s)

    def cnt_pred(mid):
        return jnp.sum((keys >= mid).astype(jnp.int32)) >= k

    thr_k = _search_max_true(cnt_pred, _KEY_NEG_INF, _KEY_POS_INF)
    kill_k = (k > 0) & (keys < thr_k)
    s1 = jnp.where(kill_k, _NEG, s)
    keys1 = jnp.where(kill_k, jnp.int32(_KEY_NEG_INF), keys)

    mx = jnp.max(s1)
    e = jnp.exp(s1 - mx)
    z = jnp.sum(e)
    target = (1.0 - p) * z

    def mass_pred(mid):
        return jnp.sum(jnp.where(keys1 <= mid, e, 0.0)) <= target

    thr_p = _search_max_true(mass_pred, _KEY_NEG_INF - 1, _KEY_POS_INF)
    s2 = jnp.where((p < 1.0) & (keys1 <= thr_p), _NEG, s1)

    cand = s2 + g
    m = jnp.max(cand)
    return jnp.min(jnp.where(cand == m, vidx, jnp.int32(_VPAD)))


def _iota_vidx():
    rowi = lax.broadcasted_iota(jnp.int32, (_NCH, _CW), 0)
    coli = lax.broadcasted_iota(jnp.int32, (_NCH, _CW), 1)
    return rowi * _CW + coli


def _step_kernel(row_ref, w_ref, g_ref, temp_ref, k_ref, p_ref,
                 tok_ref, s_scr):
    j = pl.program_id(0)
    chunk = jnp.dot(row_ref[...], w_ref[...],
                    preferred_element_type=jnp.float32)
    s_scr[pl.ds(j, 1), :] = chunk / temp_ref[0, 0]

    @pl.when(j == _NCH - 1)
    def _():
        vidx = _iota_vidx()
        s = jnp.where(vidx < _VOCAB, s_scr[...], _NEG)
        tok_ref[0, 0] = _select_token(s, g_ref[...], k_ref[0, 0],
                                      p_ref[0, 0], vidx)


def _final_kernel(rows_ref, w_ref, g_ref, temp_ref, k_ref, p_ref,
                  out_ref, tok_ref, s_scr):
    j = pl.program_id(0)
    mat = jnp.dot(rows_ref[...], w_ref[...],
                  preferred_element_type=jnp.float32)
    out_ref[...] = mat[:39, :]
    s_scr[pl.ds(j, 1), :] = mat[38:39, :] / temp_ref[0, 0]

    @pl.when(j == _NCH - 1)
    def _():
        vidx = _iota_vidx()
        s = jnp.where(vidx < _VOCAB, s_scr[...], _NEG)
        tok_ref[0, 0] = _select_token(s, g_ref[...], k_ref[0, 0],
                                      p_ref[0, 0], vidx)


_step_call = pl.pallas_call(
    _step_kernel,
    grid=(_NCH,),
    in_specs=[
        pl.BlockSpec((1, _DM), lambda j: (0, 0)),
        pl.BlockSpec((_DM, _CW), lambda j: (0, j)),
        pl.BlockSpec((_NCH, _CW), lambda j: (0, 0)),
        pl.BlockSpec(memory_space=pltpu.SMEM),
        pl.BlockSpec(memory_space=pltpu.SMEM),
        pl.BlockSpec(memory_space=pltpu.SMEM),
    ],
    out_specs=pl.BlockSpec(memory_space=pltpu.SMEM),
    out_shape=jax.ShapeDtypeStruct((1, 1), jnp.int32),
    scratch_shapes=[pltpu.VMEM((_NCH, _CW), jnp.float32)],
)

_final_call = pl.pallas_call(
    _final_kernel,
    grid=(_NCH,),
    in_specs=[
        pl.BlockSpec((40, _DM), lambda j: (0, 0)),
        pl.BlockSpec((_DM, _CW), lambda j: (0, j)),
        pl.BlockSpec((_NCH, _CW), lambda j: (0, 0)),
        pl.BlockSpec(memory_space=pltpu.SMEM),
        pl.BlockSpec(memory_space=pltpu.SMEM),
        pl.BlockSpec(memory_space=pltpu.SMEM),
    ],
    out_specs=[
        pl.BlockSpec((39, _CW), lambda j: (0, j)),
        pl.BlockSpec(memory_space=pltpu.SMEM),
    ],
    out_shape=[
        jax.ShapeDtypeStruct((39, _VOCAB), jnp.float32),
        jax.ShapeDtypeStruct((1, 1), jnp.int32),
    ],
    scratch_shapes=[pltpu.VMEM((_NCH, _CW), jnp.float32)],
)


@functools.cache
def _gumbel_table(gen_len):
    with jax.ensure_compile_time_eval():
        skey = jax.random.key(42)
        gs = []
        for _ in range(gen_len):
            skey, sub = jax.random.split(skey)
            gs.append(jax.random.gumbel(sub, (1, _VOCAB), jnp.float32))
        g = jnp.concatenate(gs, axis=0)
        g = jnp.pad(g, ((0, 0), (0, _VPAD - _VOCAB)))
        return np.asarray(g.reshape(gen_len, _NCH, _CW))


def kernel(input_ids, tensor_of_seq_len, temperature, top_k, top_p, emb, W):
    gen_len = tensor_of_seq_len.shape[1]
    g = jnp.asarray(_gumbel_table(gen_len))

    tk = top_k.astype(jnp.int32)
    tokens = []
    cur = input_ids[0, -1]
    for t in range(gen_len - 1):
        row = emb[cur][None, :]
        tok = _step_call(row, W, g[t], temperature, tk, top_p)[0, 0]
        tokens.append(tok)
        cur = tok

    ids39 = jnp.concatenate([input_ids[0], jnp.stack(tokens)], axis=0)
    rows = jnp.concatenate(
        [emb[ids39], jnp.zeros((1, _DM), jnp.float32)], axis=0)
    logits, tok_last = _final_call(rows, W, g[gen_len - 1], temperature,
                                   tk, top_p)
    ids40 = jnp.concatenate([ids39, tok_last[0]], axis=0)
    return ids40[None, :].astype(jnp.int32), logits[None, :, :]
